# baseline (speedup 1.0000x reference)
import jax
import jax.numpy as jnp
from jax.experimental import pallas as pl
from jax.experimental.pallas import tpu as pltpu

_NBUF = 6


def _pick_block_rows(rows: int) -> int:
    best = 8
    for cand in range(8, 6200, 8):
        if rows % cand == 0:
            best = cand
    return best


def _ring_copy(src, dst, row0, nrows, br, ibufs, obufs, isems, osems):
    dim = src.shape[1]

    def in_cp(i, j):
        return pltpu.make_async_copy(
            src.at[pl.ds(row0 + i * br, br)],
            ibufs.at[j, pl.ds(0, br)],
            isems.at[j],
        )

    def out_cp(i, j):
        return pltpu.make_async_copy(
            obufs.at[j, pl.ds(0, br)],
            dst.at[pl.ds(row0 + i * br, br)],
            osems.at[j],
        )

    n = nrows // br
    for i in range(min(_NBUF, n)):
        in_cp(i, i).start()
    for i in range(n):
        ji = i % _NBUF
        jo = i % _NBUF
        in_cp(i, ji).wait()
        if i >= _NBUF:
            out_cp(i - _NBUF, jo).wait()
        obufs[jo, pl.ds(0, br)] = ibufs[ji, pl.ds(0, br)]
        out_cp(i, jo).start()
        if i + _NBUF < n:
            in_cp(i + _NBUF, ji).start()
    for i in range(max(0, n - _NBUF), n):
        out_cp(i, i % _NBUF).wait()


def _body(u_in, i_in, u_out, i_out, ibufs, obufs, isems, osems):
    pid = pl.program_id(0)
    half_u = u_in.shape[0] // 2
    half_i = i_in.shape[0] // 2
    br_u = _pick_block_rows(half_u)
    br_i = _pick_block_rows(half_i)
    _ring_copy(u_in, u_out, pid * half_u, half_u, br_u,
               ibufs, obufs, isems, osems)
    _ring_copy(i_in, i_out, pid * half_i, half_i, br_i,
               ibufs, obufs, isems, osems)


def kernel(embed_user, embed_item):
    u_shape, i_shape = embed_user.shape, embed_item.shape
    embed_user = embed_user.reshape(-1, 128)
    embed_item = embed_item.reshape(-1, 128)
    dim = embed_user.shape[1]
    br_max = max(_pick_block_rows(embed_user.shape[0] // 2),
                 _pick_block_rows(embed_item.shape[0] // 2))
    out_user, out_item = pl.pallas_call(
        _body,
        grid=(2,),
        in_specs=[
            pl.BlockSpec(memory_space=pltpu.HBM),
            pl.BlockSpec(memory_space=pltpu.HBM),
        ],
        out_specs=[
            pl.BlockSpec(memory_space=pltpu.HBM),
            pl.BlockSpec(memory_space=pltpu.HBM),
        ],
        out_shape=[
            jax.ShapeDtypeStruct(embed_user.shape, embed_user.dtype),
            jax.ShapeDtypeStruct(embed_item.shape, embed_item.dtype),
        ],
        scratch_shapes=[
            pltpu.VMEM((_NBUF, br_max, dim), embed_user.dtype),
            pltpu.VMEM((_NBUF, br_max, dim), embed_user.dtype),
            pltpu.SemaphoreType.DMA((_NBUF,)),
            pltpu.SemaphoreType.DMA((_NBUF,)),
        ],
        compiler_params=pltpu.CompilerParams(
            dimension_semantics=("parallel",),
        ),
    )(embed_user, embed_item)
    return (out_user.reshape(u_shape), out_item.reshape(i_shape))


# speedup vs baseline: 7.5377x; 7.5377x over previous
import jax
import jax.numpy as jnp
from jax.experimental import pallas as pl
from jax.experimental.pallas import tpu as pltpu

_BLOCK_COLS = 8192


def _copy_body(in_ref, out_ref):
    out_ref[...] = in_ref[...]


def _copy_table(x):
    xt = x.T
    dim, cols = xt.shape
    nblk = (cols + _BLOCK_COLS - 1) // _BLOCK_COLS
    out = pl.pallas_call(
        _copy_body,
        grid=(nblk,),
        in_specs=[pl.BlockSpec((dim, _BLOCK_COLS), lambda j: (0, j))],
        out_specs=pl.BlockSpec((dim, _BLOCK_COLS), lambda j: (0, j)),
        out_shape=jax.ShapeDtypeStruct(xt.shape, xt.dtype),
        compiler_params=pltpu.CompilerParams(
            dimension_semantics=("parallel",),
        ),
    )(xt)
    return out.T


def kernel(embed_user, embed_item):
    return (_copy_table(embed_user), _copy_table(embed_item))
